# baseline (speedup 1.0000x reference)
import jax
import jax.numpy as jnp
import numpy as np
from jax.experimental import pallas as pl
from jax.experimental.pallas import tpu as pltpu

H_P, W_P = 14, 14
N_TOK = H_P * W_P
EMBED_DIM = 768
NUM_HEADS = 12
DEPTH = 12
HIDDEN_DIM = 3072
R_C, R_F = 4, 8
TAU = 1.0
EPS = 1e-6
ALPHA, BETA, GAMMA = 1.0, 0.5, 0.5
RHO = 0.5
TOK_TEMP = 0.1
B, T = 8, 16
K_TOP = max(1, int(RHO * T * N_TOK))


def _interp_coeffs(t_in, t_out):
    src = ((np.arange(t_out, dtype=np.float32) + np.float32(0.5))
           * np.float32(t_in / float(t_out)) - np.float32(0.5))
    src = np.clip(src, np.float32(0.0), np.float32(t_in - 1.0))
    lo = np.floor(src).astype(np.int32)
    hi = np.minimum(lo + 1, t_in - 1)
    w = (src - lo.astype(np.float32)).astype(np.float32)
    return lo, hi, w


I2_COEF = _interp_coeffs(T - 1, T)
I4_COEF = _interp_coeffs(T - 3, T)


def _interp_rows(e, coef):
    lo, hi, w = coef
    rows = []
    for t in range(T):
        w1 = float(np.float32(1.0) - w[t])
        rows.append(e[int(lo[t])] * w1 + e[int(hi[t])] * float(w[t]))
    return jnp.stack(rows, axis=0)


def _region_one_hot(coords, centers):
    d = jnp.sqrt(jnp.maximum(
        ((coords[:, None, :] - centers[None, :, :]) ** 2).sum(-1), 0.0))
    rid = jnp.argmin(d, axis=1)
    return (rid[None, :] == jnp.arange(centers.shape[0])[:, None]).astype(
        jnp.float32)


def _normalize(h):
    mn = jnp.min(h)
    mx = jnp.max(h)
    return (h - mn) / (mx - mn + EPS)


def _kth_largest(score, k):
    hi0 = jnp.max(score) + 1.0
    lo0 = jnp.zeros((), jnp.float32)

    def body(_, carry):
        lo, hi = carry
        mid = 0.5 * (lo + hi)
        cnt = jnp.sum((score >= mid).astype(jnp.float32))
        ge = cnt >= float(k)
        return jnp.where(ge, mid, lo), jnp.where(ge, hi, mid)

    lo, _ = jax.lax.fori_loop(0, 32, body, (lo0, hi0))
    return lo


def _fused_kernel(x_ref, oh_ref,
                  ghead_ref, gch_ref, gblock_ref,
                  mask_ref, headw_ref, chw_ref, blockw_ref, gsum_ref,
                  e1_scr, e2_scr, e4_scr, hr_scr):
    oh = oh_ref[...]
    cnt = jnp.sum(oh, axis=1, keepdims=True)

    def _h(q):
        return -jnp.sum(q * jnp.log(q + EPS), axis=1)

    s_hist = []
    for t in range(T):
        x = x_ref[0, t]
        m = jnp.max(x, axis=1, keepdims=True)
        e = jnp.exp(x - m)
        z = jnp.sum(e, axis=1, keepdims=True)
        p = e / z

        s_t = p if t == 0 else s_hist[t - 1] + p
        q1 = p if t == 0 else s_t - s_hist[t - 1]
        e1_scr[t, :] = _h(q1)

        if t == 0:
            e2_scr[0, :] = jnp.zeros((N_TOK,), jnp.float32)
        else:
            q2 = s_t * 0.5 if t == 1 else (s_t - s_hist[t - 2]) * 0.5
            e2_scr[t, :] = _h(q2)

        if t >= 3:
            q4 = s_t * 0.25 if t == 3 else (s_t - s_hist[t - 4]) * 0.25
            e4_scr[t, :] = _h(q4)

        s_hist.append(s_t)

        p_sum = jnp.dot(oh, p, preferred_element_type=jnp.float32)
        p_reg = p_sum / (cnt + EPS)
        hr_scr[t, :] = _h(p_reg)

    e1 = e1_scr[...]
    e2 = e2_scr[1:T, :]
    e4 = e4_scr[3:T, :]
    i2 = _interp_rows(e2, I2_COEF)
    i4 = _interp_rows(e4, I4_COEF)
    ht = (e1 + i2 + i4) * (1.0 / 3.0)
    ht_n = _normalize(ht)
    hr = hr_scr[...]
    hc_n = _normalize(hr[:, :R_C])
    hf_n = _normalize(hr[:, R_C:])
    hc_tok = sum(hc_n[:, r:r + 1] * oh[r:r + 1, :] for r in range(R_C))
    hf_tok = sum(hf_n[:, r:r + 1] * oh[R_C + r:R_C + r + 1, :]
                 for r in range(R_F))
    score = ALPHA * ht_n + BETA * hc_tok + GAMMA * hf_tok
    kth = _kth_largest(score, K_TOP)
    mask = jax.nn.sigmoid((score - kth) * (1.0 / TOK_TEMP))
    mask_ref[0] = mask

    head_w = jax.nn.sigmoid(ghead_ref[...])
    ch_w = jax.nn.sigmoid(gch_ref[...])
    block_w = jax.nn.sigmoid(gblock_ref[...])
    headw_ref[...] = head_w
    chw_ref[...] = ch_w
    blockw_ref[...] = block_w
    gsum = ((1.0 - jnp.mean(head_w)) + (1.0 - jnp.mean(ch_w))
            + (1.0 - jnp.mean(block_w)))
    gsum_ref[...] = jnp.reshape(gsum, (1, 1))


def kernel(token_feat, centers_coarse, centers_fine, g_head, g_ch, g_block,
           patch_coords):
    oh = jnp.concatenate([
        _region_one_hot(patch_coords, centers_coarse),
        _region_one_hot(patch_coords, centers_fine),
    ], axis=0)

    n, c = N_TOK, EMBED_DIM
    const = lambda b: (0, 0)
    mask, head_w, ch_w, block_w2, gsum = pl.pallas_call(
        _fused_kernel,
        grid=(B,),
        in_specs=[
            pl.BlockSpec((1, T, n, c), lambda b: (b, 0, 0, 0)),
            pl.BlockSpec((R_C + R_F, n), const),
            pl.BlockSpec((DEPTH, NUM_HEADS), const),
            pl.BlockSpec((DEPTH, HIDDEN_DIM), const),
            pl.BlockSpec((1, DEPTH), const),
        ],
        out_specs=[
            pl.BlockSpec((1, T, n), lambda b: (b, 0, 0)),
            pl.BlockSpec((DEPTH, NUM_HEADS), const),
            pl.BlockSpec((DEPTH, HIDDEN_DIM), const),
            pl.BlockSpec((1, DEPTH), const),
            pl.BlockSpec((1, 1), const),
        ],
        out_shape=[
            jax.ShapeDtypeStruct((B, T, n), jnp.float32),
            jax.ShapeDtypeStruct((DEPTH, NUM_HEADS), jnp.float32),
            jax.ShapeDtypeStruct((DEPTH, HIDDEN_DIM), jnp.float32),
            jax.ShapeDtypeStruct((1, DEPTH), jnp.float32),
            jax.ShapeDtypeStruct((1, 1), jnp.float32),
        ],
        scratch_shapes=[
            pltpu.VMEM((T, n), jnp.float32),
            pltpu.VMEM((T, n), jnp.float32),
            pltpu.VMEM((T, n), jnp.float32),
            pltpu.VMEM((T, R_C + R_F), jnp.float32),
        ],
        compiler_params=pltpu.CompilerParams(
            dimension_semantics=("parallel",)),
    )(token_feat, oh, g_head, g_ch, g_block.reshape(1, DEPTH))
    sparsity_token = 1.0 - mask.mean()
    l_ast = sparsity_token + gsum.reshape(())
    return (mask, head_w, ch_w, block_w2.reshape(DEPTH),
            sparsity_token, l_ast)
